# Initial kernel scaffold; baseline (speedup 1.0000x reference)
import jax
import jax.numpy as jnp
from jax.experimental import pallas as pl


def kernel(x, router_W, router_b, W1, b1, W2, b2, sW1, sb1, sW2, sb2):
    raise NotImplementedError("write your pallas kernel here")


# speedup vs baseline: 2.1852x; 2.1852x over previous
import functools

import jax
import jax.numpy as jnp
from jax.experimental import pallas as pl

DIM = 1024
E = 8
H = 2048
EP = 9
EPAD = 16
NEG = -1e30


def _gelu_exact(u):
    return 0.5 * u * (1.0 + jax.lax.erf(u * (2.0 ** -0.5)))


def _router_body(x_ref, rw_ref, rb_ref, gates_ref):
    x = x_ref[...]
    logits = jnp.dot(x, rw_ref[...], preferred_element_type=jnp.float32)
    logits = logits + rb_ref[...]
    m = jnp.max(logits, axis=1, keepdims=True)
    ex = jnp.exp(logits - m)
    probs = ex / jnp.sum(ex, axis=1, keepdims=True)
    lane = jax.lax.broadcasted_iota(jnp.int32, probs.shape, 1)
    m1 = jnp.max(probs, axis=1, keepdims=True)
    i1 = jnp.min(jnp.where(probs == m1, lane, EPAD * 4), axis=1, keepdims=True)
    probs2 = jnp.where(lane == i1, NEG, probs)
    m2 = jnp.max(probs2, axis=1, keepdims=True)
    i2 = jnp.min(jnp.where(probs2 == m2, lane, EPAD * 4), axis=1, keepdims=True)
    gates = jnp.where(lane == i1, m1, 0.0) + jnp.where(lane == i2, m2, 0.0)
    gates = gates + jnp.where(lane == E, 1.0, 0.0)
    gates_ref[...] = gates[:, :EPAD]


def _moe_body(gates_ref, x_ref, w1_ref, b1_ref, w2_ref, b2_ref, out_ref):
    e = pl.program_id(1)
    x = x_ref[...]
    h = jnp.dot(x, w1_ref[0], preferred_element_type=jnp.float32)
    h = h + b1_ref[0]
    h = _gelu_exact(h)
    y = jnp.dot(h, w2_ref[0], preferred_element_type=jnp.float32)
    y = y + b2_ref[0]
    onehot = (jax.lax.broadcasted_iota(jnp.int32, (EPAD, 1), 0) == e
              ).astype(jnp.float32)
    g = jnp.dot(gates_ref[...], onehot, preferred_element_type=jnp.float32)

    @pl.when(e == 0)
    def _():
        out_ref[...] = jnp.zeros_like(out_ref)

    out_ref[...] += y * g


def kernel(x, router_W, router_b, W1, b1, W2, b2, sW1, sb1, sW2, sb2):
    B, S, dim = x.shape
    N = B * S
    xf = x.reshape(N, dim)

    rw = jnp.zeros((dim, 128), jnp.float32).at[:, :E].set(router_W)
    rb = jnp.full((1, 128), NEG, jnp.float32).at[0, :E].set(router_b)
    TB = 512
    gates = pl.pallas_call(
        _router_body,
        grid=(N // TB,),
        in_specs=[
            pl.BlockSpec((TB, dim), lambda t: (t, 0)),
            pl.BlockSpec((dim, 128), lambda t: (0, 0)),
            pl.BlockSpec((1, 128), lambda t: (0, 0)),
        ],
        out_specs=pl.BlockSpec((TB, EPAD), lambda t: (t, 0)),
        out_shape=jax.ShapeDtypeStruct((N, EPAD), jnp.float32),
    )(xf, rw, rb)

    W1s = jnp.concatenate([W1, sW1[None]], axis=0)
    W2s = jnp.concatenate([W2, sW2[None]], axis=0)
    b1s = jnp.concatenate([b1, sb1[None]], axis=0)[:, None]
    b2s = jnp.concatenate([b2, sb2[None]], axis=0)[:, None]

    TBM = 512
    out = pl.pallas_call(
        _moe_body,
        grid=(N // TBM, EP),
        in_specs=[
            pl.BlockSpec((TBM, EPAD), lambda t, e: (t, 0)),
            pl.BlockSpec((TBM, dim), lambda t, e: (t, 0)),
            pl.BlockSpec((1, dim, H), lambda t, e: (e, 0, 0)),
            pl.BlockSpec((1, 1, H), lambda t, e: (e, 0, 0)),
            pl.BlockSpec((1, H, dim), lambda t, e: (e, 0, 0)),
            pl.BlockSpec((1, 1, dim), lambda t, e: (e, 0, 0)),
        ],
        out_specs=pl.BlockSpec((TBM, dim), lambda t, e: (t, 0)),
        out_shape=jax.ShapeDtypeStruct((N, dim), jnp.float32),
    )(gates, xf, W1s, b1s, W2s, b2s)

    return out.reshape(B, S, dim)
